# Initial kernel scaffold; baseline (speedup 1.0000x reference)
import jax
import jax.numpy as jnp
from jax.experimental import pallas as pl


def kernel(tensor, segment_ids, weight, bias):
    raise NotImplementedError("write your pallas kernel here")


# speedup vs baseline: 5.4839x; 5.4839x over previous
import functools

import jax
import jax.numpy as jnp
from jax.experimental import pallas as pl

_NUM_SEGMENTS = 64
_EPS = 1e-05
_N = 100000
_D = 128
_BR = 2000
_NB = _N // _BR


def _pass1_kernel(x_ref, seg_ref, s_ref, q_ref, m_ref):
    i = pl.program_id(0)
    x = x_ref[...]
    seg = seg_ref[0, 0, :]
    seg_iota = jax.lax.broadcasted_iota(jnp.int32, (_NUM_SEGMENTS, _BR), 0)
    one_hot_t = (seg_iota == seg[None, :]).astype(jnp.float32)
    s_part = jnp.dot(one_hot_t, x, preferred_element_type=jnp.float32)
    q_part = jnp.dot(one_hot_t, x * x, preferred_element_type=jnp.float32)
    absx = jnp.abs(x)
    rowmax = jnp.max(absx, axis=1)
    m_part = jnp.max(jnp.where(one_hot_t > 0.5, rowmax[None, :], 0.0), axis=1)
    m_part2d = jnp.broadcast_to(m_part[:, None], (_NUM_SEGMENTS, _D))

    @pl.when(i == 0)
    def _():
        s_ref[...] = jnp.zeros_like(s_ref)
        q_ref[...] = jnp.zeros_like(q_ref)
        m_ref[...] = jnp.zeros_like(m_ref)

    s_ref[...] += s_part
    q_ref[...] += q_part
    m_ref[...] = jnp.maximum(m_ref[...], m_part2d)


def _pass2_kernel(x_ref, seg_ref, s_ref, q_ref, m_ref, w_ref, b_ref, o_ref):
    m = jnp.max(m_ref[...], axis=1, keepdims=True)
    m = jnp.maximum(m, 1e-12)
    denom = jnp.sqrt(m)
    sum_t = jnp.sum(s_ref[...] / denom, axis=0, keepdims=True)
    sum_t2 = jnp.sum(q_ref[...] / m, axis=0, keepdims=True)
    mean = sum_t / _N
    var = (sum_t2 - mean * sum_t) / (_N - 1)
    invstd = jax.lax.rsqrt(var + _EPS)
    scale = w_ref[...] * invstd
    a = scale / denom
    c = b_ref[...] - mean * scale

    x = x_ref[...]
    seg = seg_ref[0, 0, :]
    seg_iota = jax.lax.broadcasted_iota(jnp.int32, (_BR, _NUM_SEGMENTS), 1)
    one_hot = (seg_iota == seg[:, None]).astype(jnp.float32)
    a_rows = jnp.dot(one_hot, a, preferred_element_type=jnp.float32)
    o_ref[...] = x * a_rows + c


@functools.partial(jax.jit, static_argnames=())
def _run(tensor, seg3d, weight, bias):
    stats_shape = jax.ShapeDtypeStruct((_NUM_SEGMENTS, _D), jnp.float32)
    s, q, m = pl.pallas_call(
        _pass1_kernel,
        grid=(_NB,),
        in_specs=[
            pl.BlockSpec((_BR, _D), lambda i: (i, 0)),
            pl.BlockSpec((1, 1, _BR), lambda i: (i, 0, 0)),
        ],
        out_specs=[
            pl.BlockSpec((_NUM_SEGMENTS, _D), lambda i: (0, 0)),
            pl.BlockSpec((_NUM_SEGMENTS, _D), lambda i: (0, 0)),
            pl.BlockSpec((_NUM_SEGMENTS, _D), lambda i: (0, 0)),
        ],
        out_shape=[stats_shape, stats_shape, stats_shape],
    )(tensor, seg3d)

    out = pl.pallas_call(
        _pass2_kernel,
        grid=(_NB,),
        in_specs=[
            pl.BlockSpec((_BR, _D), lambda i: (i, 0)),
            pl.BlockSpec((1, 1, _BR), lambda i: (i, 0, 0)),
            pl.BlockSpec((_NUM_SEGMENTS, _D), lambda i: (0, 0)),
            pl.BlockSpec((_NUM_SEGMENTS, _D), lambda i: (0, 0)),
            pl.BlockSpec((_NUM_SEGMENTS, _D), lambda i: (0, 0)),
            pl.BlockSpec((1, _D), lambda i: (0, 0)),
            pl.BlockSpec((1, _D), lambda i: (0, 0)),
        ],
        out_specs=pl.BlockSpec((_BR, _D), lambda i: (i, 0)),
        out_shape=jax.ShapeDtypeStruct((_N, _D), jnp.float32),
    )(tensor, seg3d, s, q, m, weight.reshape(1, _D), bias.reshape(1, _D))
    return out


def kernel(tensor, segment_ids, weight, bias):
    seg3d = segment_ids.astype(jnp.int32).reshape(_NB, 1, _BR)
    return _run(tensor, seg3d, weight, bias)


# speedup vs baseline: 7.6536x; 1.3956x over previous
import jax
import jax.numpy as jnp
from jax.experimental import pallas as pl

_NUM_SEGMENTS = 64
_EPS = 1e-05
_N = 100000
_D = 128
_BR = 4000
_NB = _N // _BR


def _pass1_kernel(x_ref, seg_ref, s_ref, q_ref, msq_ref):
    i = pl.program_id(0)
    x = x_ref[...]
    seg = seg_ref[0, 0, :]
    seg_iota = jax.lax.broadcasted_iota(jnp.int32, (_NUM_SEGMENTS, _BR), 0)
    one_hot_t = (seg_iota == seg[None, :]).astype(jnp.float32)
    xx = x * x
    s_part = jnp.dot(one_hot_t, x, preferred_element_type=jnp.float32)
    q_part = jnp.dot(one_hot_t, xx, preferred_element_type=jnp.float32)
    rowmaxsq = jnp.max(xx, axis=1)
    msq_part = jnp.max(one_hot_t * rowmaxsq[None, :], axis=1)
    msq_part2d = jnp.broadcast_to(msq_part[:, None], (_NUM_SEGMENTS, _D))

    @pl.when(i == 0)
    def _():
        s_ref[...] = jnp.zeros_like(s_ref)
        q_ref[...] = jnp.zeros_like(q_ref)
        msq_ref[...] = jnp.zeros_like(msq_ref)

    s_ref[...] += s_part
    q_ref[...] += q_part
    msq_ref[...] = jnp.maximum(msq_ref[...], msq_part2d)


def _pass2_kernel(x_ref, seg_ref, s_ref, q_ref, msq_ref, w_ref, b_ref, o_ref):
    m = jnp.sqrt(jnp.max(msq_ref[...], axis=1, keepdims=True))
    m = jnp.maximum(m, 1e-12)
    denom = jnp.sqrt(m)
    sum_t = jnp.sum(s_ref[...] / denom, axis=0, keepdims=True)
    sum_t2 = jnp.sum(q_ref[...] / m, axis=0, keepdims=True)
    mean = sum_t / _N
    var = (sum_t2 - mean * sum_t) / (_N - 1)
    invstd = jax.lax.rsqrt(var + _EPS)
    scale = w_ref[...] * invstd
    a = scale / denom
    c = b_ref[...] - mean * scale

    x = x_ref[...]
    seg = seg_ref[0, 0, :]
    seg_iota = jax.lax.broadcasted_iota(jnp.int32, (_BR, _NUM_SEGMENTS), 1)
    one_hot = (seg_iota == seg[:, None]).astype(jnp.float32)
    a_rows = jnp.dot(one_hot, a, preferred_element_type=jnp.float32)
    o_ref[...] = x * a_rows + c


@jax.jit
def _run(tensor, segment_ids, weight, bias):
    seg3d = segment_ids.astype(jnp.int32).reshape(_NB, 1, _BR)
    stats_shape = jax.ShapeDtypeStruct((_NUM_SEGMENTS, _D), jnp.float32)
    s, q, msq = pl.pallas_call(
        _pass1_kernel,
        grid=(_NB,),
        in_specs=[
            pl.BlockSpec((_BR, _D), lambda i: (i, 0)),
            pl.BlockSpec((1, 1, _BR), lambda i: (i, 0, 0)),
        ],
        out_specs=[
            pl.BlockSpec((_NUM_SEGMENTS, _D), lambda i: (0, 0)),
            pl.BlockSpec((_NUM_SEGMENTS, _D), lambda i: (0, 0)),
            pl.BlockSpec((_NUM_SEGMENTS, _D), lambda i: (0, 0)),
        ],
        out_shape=[stats_shape, stats_shape, stats_shape],
    )(tensor, seg3d)

    out = pl.pallas_call(
        _pass2_kernel,
        grid=(_NB,),
        in_specs=[
            pl.BlockSpec((_BR, _D), lambda i: (i, 0)),
            pl.BlockSpec((1, 1, _BR), lambda i: (i, 0, 0)),
            pl.BlockSpec((_NUM_SEGMENTS, _D), lambda i: (0, 0)),
            pl.BlockSpec((_NUM_SEGMENTS, _D), lambda i: (0, 0)),
            pl.BlockSpec((_NUM_SEGMENTS, _D), lambda i: (0, 0)),
            pl.BlockSpec((1, _D), lambda i: (0, 0)),
            pl.BlockSpec((1, _D), lambda i: (0, 0)),
        ],
        out_specs=pl.BlockSpec((_BR, _D), lambda i: (i, 0)),
        out_shape=jax.ShapeDtypeStruct((_N, _D), jnp.float32),
    )(tensor, seg3d, s, q, msq, weight.reshape(1, _D), bias.reshape(1, _D))
    return out


def kernel(tensor, segment_ids, weight, bias):
    return _run(tensor, segment_ids, weight, bias)


# speedup vs baseline: 8.1265x; 1.0618x over previous
import jax
import jax.numpy as jnp
from jax.experimental import pallas as pl

_NUM_SEGMENTS = 64
_EPS = 1e-05
_N = 100000
_D = 128
_BR = 5000
_NB = _N // _BR


def _pass1_kernel(x_ref, seg_ref, s_ref, q_ref, msq_ref):
    i = pl.program_id(0)
    x = x_ref[...]
    seg = seg_ref[0, 0, :]
    seg_iota = jax.lax.broadcasted_iota(jnp.int32, (_NUM_SEGMENTS, _BR), 0)
    one_hot_t = (seg_iota == seg[None, :]).astype(jnp.float32)
    xx = x * x
    s_part = jnp.dot(one_hot_t, x, preferred_element_type=jnp.float32)
    q_part = jnp.dot(one_hot_t, xx, preferred_element_type=jnp.float32)
    rowmaxsq = jnp.max(xx, axis=1)
    msq_part = jnp.max(one_hot_t * rowmaxsq[None, :], axis=1)
    msq_part2d = jnp.broadcast_to(msq_part[:, None], (_NUM_SEGMENTS, _D))

    @pl.when(i == 0)
    def _():
        s_ref[...] = jnp.zeros_like(s_ref)
        q_ref[...] = jnp.zeros_like(q_ref)
        msq_ref[...] = jnp.zeros_like(msq_ref)

    s_ref[...] += s_part
    q_ref[...] += q_part
    msq_ref[...] = jnp.maximum(msq_ref[...], msq_part2d)


def _pass2_kernel(x_ref, seg_ref, s_ref, q_ref, msq_ref, w_ref, b_ref, o_ref):
    m = jnp.sqrt(jnp.max(msq_ref[...], axis=1, keepdims=True))
    m = jnp.maximum(m, 1e-12)
    denom = jnp.sqrt(m)
    sum_t = jnp.sum(s_ref[...] / denom, axis=0, keepdims=True)
    sum_t2 = jnp.sum(q_ref[...] / m, axis=0, keepdims=True)
    mean = sum_t / _N
    var = (sum_t2 - mean * sum_t) / (_N - 1)
    invstd = jax.lax.rsqrt(var + _EPS)
    scale = w_ref[...] * invstd
    a = scale / denom
    c = b_ref[...] - mean * scale

    x = x_ref[...]
    seg = seg_ref[0, 0, :]
    seg_iota = jax.lax.broadcasted_iota(jnp.int32, (_BR, _NUM_SEGMENTS), 1)
    one_hot = (seg_iota == seg[:, None]).astype(jnp.float32)
    a_rows = jnp.dot(one_hot, a, preferred_element_type=jnp.float32)
    o_ref[...] = x * a_rows + c


@jax.jit
def _run(tensor, segment_ids, weight, bias):
    seg3d = segment_ids.astype(jnp.int32).reshape(_NB, 1, _BR)
    stats_shape = jax.ShapeDtypeStruct((_NUM_SEGMENTS, _D), jnp.float32)
    s, q, msq = pl.pallas_call(
        _pass1_kernel,
        grid=(_NB,),
        in_specs=[
            pl.BlockSpec((_BR, _D), lambda i: (i, 0)),
            pl.BlockSpec((1, 1, _BR), lambda i: (i, 0, 0)),
        ],
        out_specs=[
            pl.BlockSpec((_NUM_SEGMENTS, _D), lambda i: (0, 0)),
            pl.BlockSpec((_NUM_SEGMENTS, _D), lambda i: (0, 0)),
            pl.BlockSpec((_NUM_SEGMENTS, _D), lambda i: (0, 0)),
        ],
        out_shape=[stats_shape, stats_shape, stats_shape],
    )(tensor, seg3d)

    out = pl.pallas_call(
        _pass2_kernel,
        grid=(_NB,),
        in_specs=[
            pl.BlockSpec((_BR, _D), lambda i: (i, 0)),
            pl.BlockSpec((1, 1, _BR), lambda i: (i, 0, 0)),
            pl.BlockSpec((_NUM_SEGMENTS, _D), lambda i: (0, 0)),
            pl.BlockSpec((_NUM_SEGMENTS, _D), lambda i: (0, 0)),
            pl.BlockSpec((_NUM_SEGMENTS, _D), lambda i: (0, 0)),
            pl.BlockSpec((1, _D), lambda i: (0, 0)),
            pl.BlockSpec((1, _D), lambda i: (0, 0)),
        ],
        out_specs=pl.BlockSpec((_BR, _D), lambda i: (i, 0)),
        out_shape=jax.ShapeDtypeStruct((_N, _D), jnp.float32),
    )(tensor, seg3d, s, q, msq, weight.reshape(1, _D), bias.reshape(1, _D))
    return out


def kernel(tensor, segment_ids, weight, bias):
    return _run(tensor, segment_ids, weight, bias)


# speedup vs baseline: 9.2947x; 1.1437x over previous
import jax
import jax.numpy as jnp
from jax.experimental import pallas as pl

_NUM_SEGMENTS = 64
_EPS = 1e-05
_N = 100000
_D = 128
_BR = 10000
_NB = _N // _BR


def _pass1_kernel(x_ref, seg_ref, s_ref, q_ref, msq_ref):
    i = pl.program_id(0)
    x = x_ref[...]
    seg = seg_ref[0, 0, :]
    seg_iota = jax.lax.broadcasted_iota(jnp.int32, (_NUM_SEGMENTS, _BR), 0)
    one_hot_t = (seg_iota == seg[None, :]).astype(jnp.float32)
    xx = x * x
    s_part = jnp.dot(one_hot_t, x, preferred_element_type=jnp.float32)
    q_part = jnp.dot(one_hot_t, xx, preferred_element_type=jnp.float32)
    rowmaxsq = jnp.max(xx, axis=1)
    msq_part = jnp.max(one_hot_t * rowmaxsq[None, :], axis=1)
    msq_part2d = jnp.broadcast_to(msq_part[:, None], (_NUM_SEGMENTS, _D))

    @pl.when(i == 0)
    def _():
        s_ref[...] = jnp.zeros_like(s_ref)
        q_ref[...] = jnp.zeros_like(q_ref)
        msq_ref[...] = jnp.zeros_like(msq_ref)

    s_ref[...] += s_part
    q_ref[...] += q_part
    msq_ref[...] = jnp.maximum(msq_ref[...], msq_part2d)


def _pass2_kernel(x_ref, seg_ref, s_ref, q_ref, msq_ref, w_ref, b_ref, o_ref):
    m = jnp.sqrt(jnp.max(msq_ref[...], axis=1, keepdims=True))
    m = jnp.maximum(m, 1e-12)
    denom = jnp.sqrt(m)
    sum_t = jnp.sum(s_ref[...] / denom, axis=0, keepdims=True)
    sum_t2 = jnp.sum(q_ref[...] / m, axis=0, keepdims=True)
    mean = sum_t / _N
    var = (sum_t2 - mean * sum_t) / (_N - 1)
    invstd = jax.lax.rsqrt(var + _EPS)
    scale = w_ref[...] * invstd
    a = scale / denom
    c = b_ref[...] - mean * scale

    x = x_ref[...]
    seg = seg_ref[0, 0, :]
    seg_iota = jax.lax.broadcasted_iota(jnp.int32, (_BR, _NUM_SEGMENTS), 1)
    one_hot = (seg_iota == seg[:, None]).astype(jnp.float32)
    a_rows = jnp.dot(one_hot, a, preferred_element_type=jnp.float32)
    o_ref[...] = x * a_rows + c


@jax.jit
def _run(tensor, segment_ids, weight, bias):
    seg3d = segment_ids.astype(jnp.int32).reshape(_NB, 1, _BR)
    stats_shape = jax.ShapeDtypeStruct((_NUM_SEGMENTS, _D), jnp.float32)
    s, q, msq = pl.pallas_call(
        _pass1_kernel,
        grid=(_NB,),
        in_specs=[
            pl.BlockSpec((_BR, _D), lambda i: (i, 0)),
            pl.BlockSpec((1, 1, _BR), lambda i: (i, 0, 0)),
        ],
        out_specs=[
            pl.BlockSpec((_NUM_SEGMENTS, _D), lambda i: (0, 0)),
            pl.BlockSpec((_NUM_SEGMENTS, _D), lambda i: (0, 0)),
            pl.BlockSpec((_NUM_SEGMENTS, _D), lambda i: (0, 0)),
        ],
        out_shape=[stats_shape, stats_shape, stats_shape],
    )(tensor, seg3d)

    out = pl.pallas_call(
        _pass2_kernel,
        grid=(_NB,),
        in_specs=[
            pl.BlockSpec((_BR, _D), lambda i: (i, 0)),
            pl.BlockSpec((1, 1, _BR), lambda i: (i, 0, 0)),
            pl.BlockSpec((_NUM_SEGMENTS, _D), lambda i: (0, 0)),
            pl.BlockSpec((_NUM_SEGMENTS, _D), lambda i: (0, 0)),
            pl.BlockSpec((_NUM_SEGMENTS, _D), lambda i: (0, 0)),
            pl.BlockSpec((1, _D), lambda i: (0, 0)),
            pl.BlockSpec((1, _D), lambda i: (0, 0)),
        ],
        out_specs=pl.BlockSpec((_BR, _D), lambda i: (i, 0)),
        out_shape=jax.ShapeDtypeStruct((_N, _D), jnp.float32),
    )(tensor, seg3d, s, q, msq, weight.reshape(1, _D), bias.reshape(1, _D))
    return out


def kernel(tensor, segment_ids, weight, bias):
    return _run(tensor, segment_ids, weight, bias)


# speedup vs baseline: 9.4440x; 1.0161x over previous
import jax
import jax.numpy as jnp
from jax.experimental import pallas as pl

_NUM_SEGMENTS = 64
_EPS = 1e-05
_N = 100000
_D = 128
_BR = 20000
_NB = _N // _BR


def _pass1_kernel(x_ref, seg_ref, s_ref, q_ref, msq_ref):
    i = pl.program_id(0)
    x = x_ref[...]
    seg = seg_ref[0, 0, :]
    seg_iota = jax.lax.broadcasted_iota(jnp.int32, (_NUM_SEGMENTS, _BR), 0)
    one_hot_t = (seg_iota == seg[None, :]).astype(jnp.float32)
    xx = x * x
    s_part = jnp.dot(one_hot_t, x, preferred_element_type=jnp.float32)
    q_part = jnp.dot(one_hot_t, xx, preferred_element_type=jnp.float32)
    rowmaxsq = jnp.max(xx, axis=1)
    msq_part = jnp.max(one_hot_t * rowmaxsq[None, :], axis=1)
    msq_part2d = jnp.broadcast_to(msq_part[:, None], (_NUM_SEGMENTS, _D))

    @pl.when(i == 0)
    def _():
        s_ref[...] = jnp.zeros_like(s_ref)
        q_ref[...] = jnp.zeros_like(q_ref)
        msq_ref[...] = jnp.zeros_like(msq_ref)

    s_ref[...] += s_part
    q_ref[...] += q_part
    msq_ref[...] = jnp.maximum(msq_ref[...], msq_part2d)


def _pass2_kernel(x_ref, seg_ref, s_ref, q_ref, msq_ref, w_ref, b_ref, o_ref):
    m = jnp.sqrt(jnp.max(msq_ref[...], axis=1, keepdims=True))
    m = jnp.maximum(m, 1e-12)
    denom = jnp.sqrt(m)
    sum_t = jnp.sum(s_ref[...] / denom, axis=0, keepdims=True)
    sum_t2 = jnp.sum(q_ref[...] / m, axis=0, keepdims=True)
    mean = sum_t / _N
    var = (sum_t2 - mean * sum_t) / (_N - 1)
    invstd = jax.lax.rsqrt(var + _EPS)
    scale = w_ref[...] * invstd
    a = scale / denom
    c = b_ref[...] - mean * scale

    x = x_ref[...]
    seg = seg_ref[0, 0, :]
    seg_iota = jax.lax.broadcasted_iota(jnp.int32, (_BR, _NUM_SEGMENTS), 1)
    one_hot = (seg_iota == seg[:, None]).astype(jnp.float32)
    a_rows = jnp.dot(one_hot, a, preferred_element_type=jnp.float32)
    o_ref[...] = x * a_rows + c


@jax.jit
def _run(tensor, segment_ids, weight, bias):
    seg3d = segment_ids.astype(jnp.int32).reshape(_NB, 1, _BR)
    stats_shape = jax.ShapeDtypeStruct((_NUM_SEGMENTS, _D), jnp.float32)
    s, q, msq = pl.pallas_call(
        _pass1_kernel,
        grid=(_NB,),
        in_specs=[
            pl.BlockSpec((_BR, _D), lambda i: (i, 0)),
            pl.BlockSpec((1, 1, _BR), lambda i: (i, 0, 0)),
        ],
        out_specs=[
            pl.BlockSpec((_NUM_SEGMENTS, _D), lambda i: (0, 0)),
            pl.BlockSpec((_NUM_SEGMENTS, _D), lambda i: (0, 0)),
            pl.BlockSpec((_NUM_SEGMENTS, _D), lambda i: (0, 0)),
        ],
        out_shape=[stats_shape, stats_shape, stats_shape],
    )(tensor, seg3d)

    out = pl.pallas_call(
        _pass2_kernel,
        grid=(_NB,),
        in_specs=[
            pl.BlockSpec((_BR, _D), lambda i: (i, 0)),
            pl.BlockSpec((1, 1, _BR), lambda i: (i, 0, 0)),
            pl.BlockSpec((_NUM_SEGMENTS, _D), lambda i: (0, 0)),
            pl.BlockSpec((_NUM_SEGMENTS, _D), lambda i: (0, 0)),
            pl.BlockSpec((_NUM_SEGMENTS, _D), lambda i: (0, 0)),
            pl.BlockSpec((1, _D), lambda i: (0, 0)),
            pl.BlockSpec((1, _D), lambda i: (0, 0)),
        ],
        out_specs=pl.BlockSpec((_BR, _D), lambda i: (i, 0)),
        out_shape=jax.ShapeDtypeStruct((_N, _D), jnp.float32),
    )(tensor, seg3d, s, q, msq, weight.reshape(1, _D), bias.reshape(1, _D))
    return out


def kernel(tensor, segment_ids, weight, bias):
    return _run(tensor, segment_ids, weight, bias)
